# baseline (speedup 1.0000x reference)
import functools

import jax
import jax.numpy as jnp
from jax import lax
from jax.experimental import pallas as pl
from jax.experimental.pallas import tpu as pltpu
from jax.experimental.pallas import tpu_sc as plsc

NUM_EMB = 1000000
DIM = 64
PDIM = 72
N_TOK = 4096
SEQ = 50
NC = 2
NS = 16
NW = NC * NS
ROWS_PER_W = N_TOK // NW
NBUF = 8
PF = NBUF - 1


def _make_kernel():
    mesh = plsc.VectorSubcoreMesh(core_axis_name="c", subcore_axis_name="s")

    @functools.partial(
        pl.kernel,
        out_type=jax.ShapeDtypeStruct((N_TOK, SEQ, DIM), jnp.float32),
        mesh=mesh,
        scratch_types=[
            pltpu.VMEM((ROWS_PER_W, SEQ), jnp.int32),
            pltpu.VMEM((NBUF, SEQ, PDIM), jnp.float32),
        ] + [pltpu.SemaphoreType.DMA] * NBUF,
        compiler_params=pltpu.CompilerParams(use_tc_tiling_on_sc=False),
    )
    def k2(tok_hbm, emb_hbm, out_hbm, idx_v, rows_v, *sems):
        wid = lax.axis_index("s") * NC + lax.axis_index("c")
        base = wid * ROWS_PER_W
        pltpu.sync_copy(tok_hbm.at[pl.ds(base, ROWS_PER_W)], idx_v)

        def gather(r, b):
            pltpu.async_copy(emb_hbm.at[idx_v.at[r]], rows_v.at[b], sems[b])

        def drain(b):
            pltpu.make_async_copy(
                emb_hbm.at[pl.ds(0, SEQ)], rows_v.at[b], sems[b]).wait()

        def put(r, b):
            pltpu.sync_copy(rows_v.at[b, :, pl.ds(0, DIM)],
                            out_hbm.at[base + r])

        for c in range(PF):
            gather(c, c)

        def group(g0, carry):
            g = g0 * NBUF
            for b in range(NBUF):
                r = g + b
                drain(b)

                @pl.when(r + PF < ROWS_PER_W)
                def _():
                    gather(r + PF, (b + PF) % NBUF)

                put(r, b)
            return carry

        lax.fori_loop(0, ROWS_PER_W // NBUF, group, 0, unroll=False)

    return k2


_k2 = _make_kernel()


def kernel(token_ids, embeddings):
    emb128 = jnp.pad(embeddings, ((0, 0), (0, PDIM - DIM)))
    return _k2(token_ids.astype(jnp.int32), emb128)


# speedup vs baseline: 1.7192x; 1.7192x over previous
import functools

import jax
import jax.numpy as jnp
from jax import lax
from jax.experimental import pallas as pl
from jax.experimental.pallas import tpu as pltpu
from jax.experimental.pallas import tpu_sc as plsc

NUM_EMB = 1000000
DIM = 64
PDIM = 128
N_TOK = 4096
SEQ = 50
NC = 2
NS = 16
NW = NC * NS
ROWS_PER_W = N_TOK // NW
NBUF = 8
PF = NBUF - 1


def _make_kernel():
    mesh = plsc.VectorSubcoreMesh(core_axis_name="c", subcore_axis_name="s")

    @functools.partial(
        pl.kernel,
        out_type=jax.ShapeDtypeStruct((N_TOK, SEQ, DIM), jnp.float32),
        mesh=mesh,
        scratch_types=[
            pltpu.VMEM((ROWS_PER_W, SEQ), jnp.int32),
            pltpu.VMEM((NBUF, SEQ, PDIM), jnp.float32),
        ] + [pltpu.SemaphoreType.DMA] * NBUF,
        compiler_params=pltpu.CompilerParams(use_tc_tiling_on_sc=False),
    )
    def k2(tok_hbm, emb_hbm, out_hbm, idx_v, rows_v, *sems):
        wid = lax.axis_index("s") * NC + lax.axis_index("c")
        base = wid * ROWS_PER_W
        pltpu.sync_copy(tok_hbm.at[pl.ds(base, ROWS_PER_W)], idx_v)

        def gather(r, b):
            pltpu.async_copy(emb_hbm.at[idx_v.at[r]], rows_v.at[b], sems[b])

        def drain(b):
            pltpu.make_async_copy(
                emb_hbm.at[pl.ds(0, SEQ)], rows_v.at[b], sems[b]).wait()

        def put(r, b):
            pltpu.sync_copy(rows_v.at[b, :, pl.ds(0, DIM)],
                            out_hbm.at[base + r])

        for c in range(PF):
            gather(c, c)

        def group(g0, carry):
            g = g0 * NBUF
            for b in range(NBUF):
                r = g + b
                drain(b)

                @pl.when(r + PF < ROWS_PER_W)
                def _():
                    gather(r + PF, (b + PF) % NBUF)

                put(r, b)
            return carry

        lax.fori_loop(0, ROWS_PER_W // NBUF, group, 0, unroll=False)

    return k2


_k2 = _make_kernel()


def kernel(token_ids, embeddings):
    emb128 = jnp.pad(embeddings, ((0, 0), (0, PDIM - DIM)))
    return _k2(token_ids.astype(jnp.int32), emb128)
